# baseline (speedup 1.0000x reference)
import functools

import jax
import jax.numpy as jnp
from jax import lax
from jax.experimental import pallas as pl
from jax.experimental.pallas import tpu as pltpu
from jax.experimental.pallas import tpu_sc as plsc

BATCH = 16384
FACTORS = 32

_INFO = plsc.get_sparse_core_info()
_NC = _INFO.num_cores
_NS = _INFO.num_subcores
_NW = _NC * _NS
_BPW = BATCH // _NW
_CHUNK = 128
_NCHUNK = _BPW // _CHUNK


def _sc_gather_t(user_idx, item_idx, uemb_t, iemb_t):
    mesh = plsc.VectorSubcoreMesh(core_axis_name="c", subcore_axis_name="s")

    @functools.partial(
        pl.kernel,
        mesh=mesh,
        out_type=[
            jax.ShapeDtypeStruct((FACTORS, BATCH), jnp.float32),
            jax.ShapeDtypeStruct((FACTORS, BATCH), jnp.float32),
        ],
        scratch_types=[
            pltpu.VMEM((_BPW,), jnp.int32),
            pltpu.VMEM((_BPW,), jnp.int32),
            pltpu.VMEM((FACTORS, _BPW), jnp.float32),
            pltpu.VMEM((FACTORS, _BPW), jnp.float32),
            pltpu.SemaphoreType.DMA,
        ],
        compiler_params=pltpu.CompilerParams(use_tc_tiling_on_sc=False),
    )
    def k(uidx_hbm, iidx_hbm, ut_hbm, it_hbm, uo_hbm, vo_hbm,
          uidx_v, iidx_v, ubuf, ibuf, sem):
        wid = lax.axis_index("s") * _NC + lax.axis_index("c")
        base = wid * _BPW
        pltpu.sync_copy(uidx_hbm.at[pl.ds(base, _BPW)], uidx_v)
        pltpu.sync_copy(iidx_hbm.at[pl.ds(base, _BPW)], iidx_v)
        pend = []
        for c in range(_NCHUNK):
            sl = pl.ds(c * _CHUNK, _CHUNK)
            for f in range(FACTORS):
                pend.append(pltpu.async_copy(
                    ut_hbm.at[f].at[uidx_v.at[sl]], ubuf.at[f].at[sl], sem))
                pend.append(pltpu.async_copy(
                    it_hbm.at[f].at[iidx_v.at[sl]], ibuf.at[f].at[sl], sem))
        for cp in pend:
            cp.wait()
        osl = pl.ds(base, _BPW)
        pltpu.sync_copy(ubuf, uo_hbm.at[:, osl])
        pltpu.sync_copy(ibuf, vo_hbm.at[:, osl])

    return k(user_idx, item_idx, uemb_t, iemb_t)


_BN = 2048
_G = BATCH // _BN


def _mlp_body(u_ref, v_ref, w1t_ref, b1_ref, w2t_ref, b2_ref, w3t_ref,
              b3_ref, w4t_ref, b4_ref, o_ref):
    f32 = jnp.float32
    w1t = w1t_ref[...]
    h = (jnp.dot(w1t[:, :FACTORS], u_ref[...], preferred_element_type=f32)
         + jnp.dot(w1t[:, FACTORS:], v_ref[...], preferred_element_type=f32)
         + b1_ref[...])
    h = jnp.maximum(h, 0.0)
    h = jnp.dot(w2t_ref[...], h, preferred_element_type=f32) + b2_ref[...]
    h = jnp.maximum(h, 0.0)
    h = jnp.dot(w3t_ref[...], h, preferred_element_type=f32) + b3_ref[...]
    h = jnp.maximum(h, 0.0)
    s = jnp.dot(w4t_ref[...], h, preferred_element_type=f32) + b4_ref[...]
    o_ref[...] = jax.nn.sigmoid(s)


def _mlp_t(ut, vt, W1, b1, W2, b2, W3, b3, W4, b4):
    out = pl.pallas_call(
        _mlp_body,
        grid=(_G,),
        in_specs=[
            pl.BlockSpec((FACTORS, _BN), lambda i: (0, i)),
            pl.BlockSpec((FACTORS, _BN), lambda i: (0, i)),
            pl.BlockSpec((64, 64), lambda i: (0, 0)),
            pl.BlockSpec((64, 1), lambda i: (0, 0)),
            pl.BlockSpec((32, 64), lambda i: (0, 0)),
            pl.BlockSpec((32, 1), lambda i: (0, 0)),
            pl.BlockSpec((16, 32), lambda i: (0, 0)),
            pl.BlockSpec((16, 1), lambda i: (0, 0)),
            pl.BlockSpec((1, 16), lambda i: (0, 0)),
            pl.BlockSpec((1, 1), lambda i: (0, 0)),
        ],
        out_specs=pl.BlockSpec((1, _BN), lambda i: (0, i)),
        out_shape=jax.ShapeDtypeStruct((1, BATCH), jnp.float32),
    )(ut, vt, W1.T, b1.reshape(64, 1), W2.T, b2.reshape(32, 1),
      W3.T, b3.reshape(16, 1), W4.T, b4.reshape(1, 1))
    return out.reshape(BATCH)


def kernel(user_input, item_input, user_emb, item_emb,
           W1, b1, W2, b2, W3, b3, W4, b4):
    ut, vt = _sc_gather_t(user_input, item_input, user_emb.T, item_emb.T)
    return _mlp_t(ut, vt, W1, b1, W2, b2, W3, b3, W4, b4)


# speedup vs baseline: 7.2115x; 7.2115x over previous
import functools

import jax
import jax.numpy as jnp
from jax import lax
from jax.experimental import pallas as pl
from jax.experimental.pallas import tpu as pltpu
from jax.experimental.pallas import tpu_sc as plsc

BATCH = 16384
FACTORS = 32
NUM_USERS = 1000000
NUM_ITEMS = 100000

_BKC = 8192
_BKR = _BKC // 4

_INFO = plsc.get_sparse_core_info()
_NC = _INFO.num_cores
_NS = _INFO.num_subcores
_NW = _NC * _NS
_BPW = BATCH // _NW
_CHUNK = 128
_NCHUNK = _BPW // _CHUNK
_L = _INFO.num_lanes


def _repack_body(in_ref, o_ref):
    r = jax.lax.broadcasted_iota(jnp.int32, (FACTORS, FACTORS), 0)
    c = jax.lax.broadcasted_iota(jnp.int32, (FACTORS, FACTORS), 1)
    eye = jnp.where(r == c, 1.0, 0.0).astype(jnp.float32)
    for k in range(4):
        o_ref[:, 32 * k:32 * (k + 1)] = jax.lax.dot_general(
            in_ref[:, _BKR * k:_BKR * (k + 1)], eye,
            (((0,), (0,)), ((), ())),
            preferred_element_type=jnp.float32,
            precision=jax.lax.Precision.DEFAULT)


def _repack(table_t, n_rows):
    grid = -(-n_rows // _BKC)
    return pl.pallas_call(
        _repack_body,
        grid=(grid,),
        in_specs=[pl.BlockSpec((FACTORS, _BKC), lambda i: (0, i))],
        out_specs=pl.BlockSpec((_BKR, 128), lambda i: (i, 0)),
        out_shape=jax.ShapeDtypeStruct((grid * _BKR, 128), jnp.float32),
        compiler_params=pltpu.CompilerParams(
            fuse_transposed_lhs_in_matmul=True),
    )(table_t)


def _sc_gather_packed(user_idx, item_idx, up, ip):
    mesh = plsc.VectorSubcoreMesh(core_axis_name="c", subcore_axis_name="s")

    @functools.partial(
        pl.kernel,
        mesh=mesh,
        out_type=[
            jax.ShapeDtypeStruct((BATCH, 128), jnp.float32),
            jax.ShapeDtypeStruct((BATCH, 128), jnp.float32),
        ],
        scratch_types=[
            pltpu.VMEM((_BPW,), jnp.int32),
            pltpu.VMEM((_BPW,), jnp.int32),
            pltpu.VMEM((_CHUNK, 128), jnp.float32),
            pltpu.VMEM((_CHUNK, 128), jnp.float32),
            pltpu.VMEM((_CHUNK, 128), jnp.float32),
            pltpu.VMEM((_CHUNK, 128), jnp.float32),
            pltpu.SemaphoreType.DMA,
            pltpu.SemaphoreType.DMA,
        ],
    )
    def k(uidx_hbm, iidx_hbm, up_hbm, ip_hbm, u_out, v_out,
          uj_v, ij_v, ub0, ub1, ib0, ib1, gsem, wsem):
        wid = lax.axis_index("s") * _NC + lax.axis_index("c")
        base = wid * _BPW
        pltpu.sync_copy(uidx_hbm.at[pl.ds(base, _BPW)], uj_v)
        pltpu.sync_copy(iidx_hbm.at[pl.ds(base, _BPW)], ij_v)
        for i in range(_BPW // _L):
            sl = pl.ds(i * _L, _L)
            u = uj_v[sl]
            uj_v[sl] = lax.shift_left(lax.shift_right_logical(u, 13), 11) + (u & 2047)
            v = ij_v[sl]
            ij_v[sl] = lax.shift_left(lax.shift_right_logical(v, 13), 11) + (v & 2047)
        ubufs, ibufs = (ub0, ub1), (ib0, ib1)
        uwb = [None, None]
        iwb = [None, None]
        for c in range(_NCHUNK):
            sl = pl.ds(c * _CHUNK, _CHUNK)
            b = c % 2
            if uwb[b] is not None:
                uwb[b].wait()
                iwb[b].wait()
            gu = pltpu.async_copy(up_hbm.at[uj_v.at[sl]], ubufs[b], gsem)
            gi = pltpu.async_copy(ip_hbm.at[ij_v.at[sl]], ibufs[b], gsem)
            gu.wait()
            gi.wait()
            osl = pl.ds(base + c * _CHUNK, _CHUNK)
            uwb[b] = pltpu.async_copy(ubufs[b], u_out.at[osl], wsem)
            iwb[b] = pltpu.async_copy(ibufs[b], v_out.at[osl], wsem)
        for b in range(2):
            if uwb[b] is not None:
                uwb[b].wait()
                iwb[b].wait()

    return k(user_idx, item_idx, up, ip)


_BM = 2048
_G = BATCH // _BM


def _select_group(x128, sel):
    out = jnp.zeros((x128.shape[0], FACTORS), jnp.float32)
    for kk in range(4):
        out = out + jnp.where(sel == kk,
                              x128[:, kk * FACTORS:(kk + 1) * FACTORS], 0.0)
    return out


def _mlp_body(u_ref, v_ref, usel_ref, isel_ref, w1_ref, b1_ref, w2_ref,
              b2_ref, w3_ref, b3_ref, w4_ref, b4_ref, o_ref):
    f32 = jnp.float32
    usel = (usel_ref[0] >> 11) & 3
    isel = (isel_ref[0] >> 11) & 3
    u = _select_group(u_ref[...], usel)
    v = _select_group(v_ref[...], isel)
    w1 = w1_ref[...]
    h = (jnp.dot(u, w1[:FACTORS], preferred_element_type=f32)
         + jnp.dot(v, w1[FACTORS:], preferred_element_type=f32)
         + b1_ref[...])
    h = jnp.maximum(h, 0.0)
    h = jnp.dot(h, w2_ref[...], preferred_element_type=f32) + b2_ref[...]
    h = jnp.maximum(h, 0.0)
    h = jnp.dot(h, w3_ref[...], preferred_element_type=f32) + b3_ref[...]
    h = jnp.maximum(h, 0.0)
    s = jnp.sum(h * w4_ref[...], axis=1, keepdims=True) + b4_ref[...]
    o_ref[...] = jax.nn.sigmoid(s)


def _mlp(u128, v128, uidx3, iidx3, W1, b1, W2, b2, W3, b3, W4, b4):
    out = pl.pallas_call(
        _mlp_body,
        grid=(_G,),
        in_specs=[
            pl.BlockSpec((_BM, 128), lambda i: (i, 0)),
            pl.BlockSpec((_BM, 128), lambda i: (i, 0)),
            pl.BlockSpec((1, _BM, 1), lambda i: (i, 0, 0)),
            pl.BlockSpec((1, _BM, 1), lambda i: (i, 0, 0)),
            pl.BlockSpec((64, 64), lambda i: (0, 0)),
            pl.BlockSpec((1, 64), lambda i: (0, 0)),
            pl.BlockSpec((64, 32), lambda i: (0, 0)),
            pl.BlockSpec((1, 32), lambda i: (0, 0)),
            pl.BlockSpec((32, 16), lambda i: (0, 0)),
            pl.BlockSpec((1, 16), lambda i: (0, 0)),
            pl.BlockSpec((1, 16), lambda i: (0, 0)),
            pl.BlockSpec((1, 1), lambda i: (0, 0)),
        ],
        out_specs=pl.BlockSpec((_BM, 1), lambda i: (i, 0)),
        out_shape=jax.ShapeDtypeStruct((BATCH, 1), jnp.float32),
    )(u128, v128, uidx3, iidx3, W1, b1.reshape(1, 64), W2, b2.reshape(1, 32),
      W3, b3.reshape(1, 16), W4.reshape(1, 16), b4.reshape(1, 1))
    return jnp.squeeze(out, axis=-1)


def kernel(user_input, item_input, user_emb, item_emb,
           W1, b1, W2, b2, W3, b3, W4, b4):
    up = _repack(user_emb.T, NUM_USERS)
    ip = _repack(item_emb.T, NUM_ITEMS)
    u128, v128 = _sc_gather_packed(user_input, item_input, up, ip)
    uidx3 = user_input.reshape(_G, _BM, 1)
    iidx3 = item_input.reshape(_G, _BM, 1)
    return _mlp(u128, v128, uidx3, iidx3, W1, b1, W2, b2, W3, b3, W4, b4)


# speedup vs baseline: 9.3228x; 1.2928x over previous
import functools

import jax
import jax.numpy as jnp
from jax import lax
from jax.experimental import pallas as pl
from jax.experimental.pallas import tpu as pltpu
from jax.experimental.pallas import tpu_sc as plsc

BATCH = 16384
FACTORS = 32
NUM_USERS = 1000000
NUM_ITEMS = 100000

_BKC = 8192
_BKR = _BKC // 4

_INFO = plsc.get_sparse_core_info()
_NC = _INFO.num_cores
_NS = _INFO.num_subcores
_NW = _NC * _NS
_BPW = BATCH // _NW
_CHUNK = 128
_NCHUNK = _BPW // _CHUNK
_L = _INFO.num_lanes


def _repack_body(in_ref, o_ref):
    r = lax.broadcasted_iota(jnp.int32, (FACTORS, FACTORS), 0)
    c = lax.broadcasted_iota(jnp.int32, (FACTORS, FACTORS), 1)
    eye = jnp.where(r == c, 1.0, 0.0).astype(jnp.bfloat16)
    x = in_ref[...].astype(jnp.bfloat16)
    for k in range(4):
        o_ref[:, 32 * k:32 * (k + 1)] = lax.dot_general(
            x[:, _BKR * k:_BKR * (k + 1)], eye,
            (((0,), (0,)), ((), ())),
            preferred_element_type=jnp.float32)


def _repack(table_t, n_rows):
    grid = -(-n_rows // _BKC)
    return pl.pallas_call(
        _repack_body,
        grid=(grid,),
        in_specs=[pl.BlockSpec((FACTORS, _BKC), lambda i: (0, i))],
        out_specs=pl.BlockSpec((_BKR, 128), lambda i: (i, 0)),
        out_shape=jax.ShapeDtypeStruct((grid * _BKR, 128), jnp.float32),
        compiler_params=pltpu.CompilerParams(
            fuse_transposed_lhs_in_matmul=True),
    )(table_t)


def _sc_gather_packed(user_idx, item_idx, up, ip):
    mesh = plsc.VectorSubcoreMesh(core_axis_name="c", subcore_axis_name="s")

    @functools.partial(
        pl.kernel,
        mesh=mesh,
        out_type=[
            jax.ShapeDtypeStruct((BATCH, 128), jnp.float32),
            jax.ShapeDtypeStruct((BATCH, 128), jnp.float32),
        ],
        scratch_types=[
            pltpu.VMEM((_BPW,), jnp.int32),
            pltpu.VMEM((_BPW,), jnp.int32),
            pltpu.VMEM((_CHUNK, 128), jnp.float32),
            pltpu.VMEM((_CHUNK, 128), jnp.float32),
            pltpu.VMEM((_CHUNK, 128), jnp.float32),
            pltpu.VMEM((_CHUNK, 128), jnp.float32),
            pltpu.SemaphoreType.DMA,
            pltpu.SemaphoreType.DMA,
        ],
    )
    def k(uidx_hbm, iidx_hbm, up_hbm, ip_hbm, u_out, v_out,
          uj_v, ij_v, ub0, ub1, ib0, ib1, gsem, wsem):
        wid = lax.axis_index("s") * _NC + lax.axis_index("c")
        base = wid * _BPW
        pltpu.sync_copy(uidx_hbm.at[pl.ds(base, _BPW)], uj_v)
        pltpu.sync_copy(iidx_hbm.at[pl.ds(base, _BPW)], ij_v)
        for i in range(_BPW // _L):
            sl = pl.ds(i * _L, _L)
            u = uj_v[sl]
            uj_v[sl] = lax.shift_left(lax.shift_right_logical(u, 13), 11) + (u & 2047)
            v = ij_v[sl]
            ij_v[sl] = lax.shift_left(lax.shift_right_logical(v, 13), 11) + (v & 2047)
        ubufs, ibufs = (ub0, ub1), (ib0, ib1)
        uwb = [None, None]
        iwb = [None, None]
        for c in range(_NCHUNK):
            sl = pl.ds(c * _CHUNK, _CHUNK)
            b = c % 2
            if uwb[b] is not None:
                uwb[b].wait()
                iwb[b].wait()
            gu = pltpu.async_copy(up_hbm.at[uj_v.at[sl]], ubufs[b], gsem)
            gi = pltpu.async_copy(ip_hbm.at[ij_v.at[sl]], ibufs[b], gsem)
            gu.wait()
            gi.wait()
            osl = pl.ds(base + c * _CHUNK, _CHUNK)
            uwb[b] = pltpu.async_copy(ubufs[b], u_out.at[osl], wsem)
            iwb[b] = pltpu.async_copy(ibufs[b], v_out.at[osl], wsem)
        for b in range(2):
            if uwb[b] is not None:
                uwb[b].wait()
                iwb[b].wait()

    return k(user_idx, item_idx, up, ip)


_BM = 2048
_G = BATCH // _BM


def _mask_groups(x128, sel):
    cg = lax.broadcasted_iota(jnp.int32, (1, 128), 1) >> 5
    return jnp.where(cg == sel, x128, 0.0).astype(jnp.bfloat16)


def _mlp_body(u_ref, v_ref, uidx_ref, iidx_ref, w1u_ref, w1v_ref, b1_ref,
              w2_ref, b2_ref, w3_ref, b3_ref, w4_ref, b4_ref, o_ref):
    f32 = jnp.float32
    usel = (uidx_ref[...] >> 11) & 3
    isel = (iidx_ref[...] >> 11) & 3
    u = _mask_groups(u_ref[...], usel)
    v = _mask_groups(v_ref[...], isel)
    h = (jnp.dot(u, w1u_ref[...], preferred_element_type=f32)
         + jnp.dot(v, w1v_ref[...], preferred_element_type=f32)
         + b1_ref[...])
    h = jnp.maximum(h, 0.0).astype(jnp.bfloat16)
    h = jnp.dot(h, w2_ref[...], preferred_element_type=f32) + b2_ref[...]
    h = jnp.maximum(h, 0.0).astype(jnp.bfloat16)
    h = jnp.dot(h, w3_ref[...], preferred_element_type=f32) + b3_ref[...]
    h = jnp.maximum(h, 0.0)
    s = jnp.sum(h * w4_ref[...], axis=1, keepdims=True) + b4_ref[...]
    o_ref[...] = jax.nn.sigmoid(s)


def _mlp(u128, v128, uidx2, iidx2, W1, b1, W2, b2, W3, b3, W4, b4):
    bf16 = jnp.bfloat16
    w1u = jnp.concatenate([W1[:FACTORS]] * 4, axis=0).astype(bf16)
    w1v = jnp.concatenate([W1[FACTORS:]] * 4, axis=0).astype(bf16)
    out = pl.pallas_call(
        _mlp_body,
        grid=(_G,),
        in_specs=[
            pl.BlockSpec((_BM, 128), lambda i: (i, 0)),
            pl.BlockSpec((_BM, 128), lambda i: (i, 0)),
            pl.BlockSpec((_BM, 1), lambda i: (i, 0)),
            pl.BlockSpec((_BM, 1), lambda i: (i, 0)),
            pl.BlockSpec((128, 64), lambda i: (0, 0)),
            pl.BlockSpec((128, 64), lambda i: (0, 0)),
            pl.BlockSpec((1, 64), lambda i: (0, 0)),
            pl.BlockSpec((64, 32), lambda i: (0, 0)),
            pl.BlockSpec((1, 32), lambda i: (0, 0)),
            pl.BlockSpec((32, 16), lambda i: (0, 0)),
            pl.BlockSpec((1, 16), lambda i: (0, 0)),
            pl.BlockSpec((1, 16), lambda i: (0, 0)),
            pl.BlockSpec((1, 1), lambda i: (0, 0)),
        ],
        out_specs=pl.BlockSpec((_BM, 1), lambda i: (i, 0)),
        out_shape=jax.ShapeDtypeStruct((BATCH, 1), jnp.float32),
    )(u128, v128, uidx2, iidx2, w1u, w1v, b1.reshape(1, 64),
      W2.astype(bf16), b2.reshape(1, 32), W3.astype(bf16), b3.reshape(1, 16),
      W4.reshape(1, 16), b4.reshape(1, 1))
    return jnp.squeeze(out, axis=-1)


def kernel(user_input, item_input, user_emb, item_emb,
           W1, b1, W2, b2, W3, b3, W4, b4):
    up = _repack(user_emb.T, NUM_USERS)
    ip = _repack(item_emb.T, NUM_ITEMS)
    u128, v128 = _sc_gather_packed(user_input, item_input, up, ip)
    uidx2 = user_input.reshape(BATCH, 1)
    iidx2 = item_input.reshape(BATCH, 1)
    return _mlp(u128, v128, uidx2, iidx2, W1, b1, W2, b2, W3, b3, W4, b4)


# speedup vs baseline: 13.5141x; 1.4496x over previous
import functools

import jax
import jax.numpy as jnp
from jax import lax
from jax.experimental import pallas as pl
from jax.experimental.pallas import tpu as pltpu
from jax.experimental.pallas import tpu_sc as plsc

BATCH = 16384
FACTORS = 32
NUM_USERS = 1000000
NUM_ITEMS = 100000

_BKC = 16384
_BKR = _BKC // 4
_SHC = 14
_SHR = 12

_INFO = plsc.get_sparse_core_info()
_NC = _INFO.num_cores
_NS = _INFO.num_subcores
_NW = _NC * _NS
_BPW = BATCH // _NW
_CHUNK = 128
_NCHUNK = _BPW // _CHUNK
_L = _INFO.num_lanes


def _repack_body(in_ref, o_ref):
    r = lax.broadcasted_iota(jnp.int32, (FACTORS, 128), 0)
    c = lax.broadcasted_iota(jnp.int32, (FACTORS, 128), 1)
    x = in_ref[...].astype(jnp.bfloat16)
    acc = None
    for k in range(4):
        eye_k = jnp.where((c - 32 * k) == r, 1.0, 0.0).astype(jnp.bfloat16)
        part = lax.dot_general(
            x[:, _BKR * k:_BKR * (k + 1)], eye_k,
            (((0,), (0,)), ((), ())),
            preferred_element_type=jnp.float32)
        acc = part if acc is None else acc + part
    o_ref[...] = acc


def _repack(table_t, n_rows):
    grid = -(-n_rows // _BKC)
    return pl.pallas_call(
        _repack_body,
        grid=(grid,),
        in_specs=[pl.BlockSpec((FACTORS, _BKC), lambda i: (0, i))],
        out_specs=pl.BlockSpec((_BKR, 128), lambda i: (i, 0)),
        out_shape=jax.ShapeDtypeStruct((grid * _BKR, 128), jnp.float32),
        compiler_params=pltpu.CompilerParams(
            fuse_transposed_lhs_in_matmul=True),
    )(table_t)


def _sc_gather_packed(user_idx, item_idx, up, ip):
    mesh = plsc.VectorSubcoreMesh(core_axis_name="c", subcore_axis_name="s")

    @functools.partial(
        pl.kernel,
        mesh=mesh,
        out_type=[
            jax.ShapeDtypeStruct((BATCH, 128), jnp.float32),
            jax.ShapeDtypeStruct((BATCH, 128), jnp.float32),
        ],
        scratch_types=[
            pltpu.VMEM((_BPW,), jnp.int32),
            pltpu.VMEM((_BPW,), jnp.int32),
            pltpu.VMEM((_CHUNK, 128), jnp.float32),
            pltpu.VMEM((_CHUNK, 128), jnp.float32),
            pltpu.VMEM((_CHUNK, 128), jnp.float32),
            pltpu.VMEM((_CHUNK, 128), jnp.float32),
            pltpu.SemaphoreType.DMA,
            pltpu.SemaphoreType.DMA,
        ],
    )
    def k(uidx_hbm, iidx_hbm, up_hbm, ip_hbm, u_out, v_out,
          uj_v, ij_v, ub0, ub1, ib0, ib1, gsem, wsem):
        wid = lax.axis_index("s") * _NC + lax.axis_index("c")
        base = wid * _BPW
        pltpu.sync_copy(uidx_hbm.at[pl.ds(base, _BPW)], uj_v)
        pltpu.sync_copy(iidx_hbm.at[pl.ds(base, _BPW)], ij_v)
        for i in range(_BPW // _L):
            sl = pl.ds(i * _L, _L)
            u = uj_v[sl]
            uj_v[sl] = (lax.shift_left(lax.shift_right_logical(u, _SHC), _SHR)
                        + (u & (_BKR - 1)))
            v = ij_v[sl]
            ij_v[sl] = (lax.shift_left(lax.shift_right_logical(v, _SHC), _SHR)
                        + (v & (_BKR - 1)))
        ubufs, ibufs = (ub0, ub1), (ib0, ib1)
        uwb = [None, None]
        iwb = [None, None]
        for c in range(_NCHUNK):
            sl = pl.ds(c * _CHUNK, _CHUNK)
            b = c % 2
            if uwb[b] is not None:
                uwb[b].wait()
                iwb[b].wait()
            gu = pltpu.async_copy(up_hbm.at[uj_v.at[sl]], ubufs[b], gsem)
            gi = pltpu.async_copy(ip_hbm.at[ij_v.at[sl]], ibufs[b], gsem)
            gu.wait()
            gi.wait()
            osl = pl.ds(base + c * _CHUNK, _CHUNK)
            uwb[b] = pltpu.async_copy(ubufs[b], u_out.at[osl], wsem)
            iwb[b] = pltpu.async_copy(ibufs[b], v_out.at[osl], wsem)
        for b in range(2):
            if uwb[b] is not None:
                uwb[b].wait()
                iwb[b].wait()

    return k(user_idx, item_idx, up, ip)


_BM = 2048
_G = BATCH // _BM


def _mask_groups(x128, sel):
    cg = lax.broadcasted_iota(jnp.int32, (1, 128), 1) >> 5
    return jnp.where(cg == sel, x128, 0.0).astype(jnp.bfloat16)


def _mlp_body(u_ref, v_ref, uidx_ref, iidx_ref, w1u_ref, w1v_ref, b1_ref,
              w2_ref, b2_ref, w3_ref, b3_ref, w4_ref, b4_ref, o_ref):
    f32 = jnp.float32
    usel = (uidx_ref[...] >> _SHR) & 3
    isel = (iidx_ref[...] >> _SHR) & 3
    u = _mask_groups(u_ref[...], usel)
    v = _mask_groups(v_ref[...], isel)
    h = (jnp.dot(u, w1u_ref[...], preferred_element_type=f32)
         + jnp.dot(v, w1v_ref[...], preferred_element_type=f32)
         + b1_ref[...])
    h = jnp.maximum(h, 0.0).astype(jnp.bfloat16)
    h = jnp.dot(h, w2_ref[...], preferred_element_type=f32) + b2_ref[...]
    h = jnp.maximum(h, 0.0).astype(jnp.bfloat16)
    h = jnp.dot(h, w3_ref[...], preferred_element_type=f32) + b3_ref[...]
    h = jnp.maximum(h, 0.0)
    s = jnp.sum(h * w4_ref[...], axis=1, keepdims=True) + b4_ref[...]
    o_ref[...] = jax.nn.sigmoid(s)


def _mlp(u128, v128, uidx2, iidx2, W1, b1, W2, b2, W3, b3, W4, b4):
    bf16 = jnp.bfloat16
    w1u = jnp.concatenate([W1[:FACTORS]] * 4, axis=0).astype(bf16)
    w1v = jnp.concatenate([W1[FACTORS:]] * 4, axis=0).astype(bf16)
    out = pl.pallas_call(
        _mlp_body,
        grid=(_G,),
        in_specs=[
            pl.BlockSpec((_BM, 128), lambda i: (i, 0)),
            pl.BlockSpec((_BM, 128), lambda i: (i, 0)),
            pl.BlockSpec((_BM, 1), lambda i: (i, 0)),
            pl.BlockSpec((_BM, 1), lambda i: (i, 0)),
            pl.BlockSpec((128, 64), lambda i: (0, 0)),
            pl.BlockSpec((128, 64), lambda i: (0, 0)),
            pl.BlockSpec((1, 64), lambda i: (0, 0)),
            pl.BlockSpec((64, 32), lambda i: (0, 0)),
            pl.BlockSpec((1, 32), lambda i: (0, 0)),
            pl.BlockSpec((32, 16), lambda i: (0, 0)),
            pl.BlockSpec((1, 16), lambda i: (0, 0)),
            pl.BlockSpec((1, 16), lambda i: (0, 0)),
            pl.BlockSpec((1, 1), lambda i: (0, 0)),
        ],
        out_specs=pl.BlockSpec((_BM, 1), lambda i: (i, 0)),
        out_shape=jax.ShapeDtypeStruct((BATCH, 1), jnp.float32),
    )(u128, v128, uidx2, iidx2, w1u, w1v, b1.reshape(1, 64),
      W2.astype(bf16), b2.reshape(1, 32), W3.astype(bf16), b3.reshape(1, 16),
      W4.reshape(1, 16), b4.reshape(1, 1))
    return jnp.squeeze(out, axis=-1)


def kernel(user_input, item_input, user_emb, item_emb,
           W1, b1, W2, b2, W3, b3, W4, b4):
    up = _repack(user_emb.T, NUM_USERS)
    ip = _repack(item_emb.T, NUM_ITEMS)
    u128, v128 = _sc_gather_packed(user_input, item_input, up, ip)
    uidx2 = user_input.reshape(BATCH, 1)
    iidx2 = item_input.reshape(BATCH, 1)
    return _mlp(u128, v128, uidx2, iidx2, W1, b1, W2, b2, W3, b3, W4, b4)
